# baseline (speedup 1.0000x reference)
import functools
import jax
import jax.numpy as jnp
from jax import lax
from jax.experimental import pallas as pl
from jax.experimental.pallas import tpu as pltpu
from jax.experimental.pallas import tpu_sc as plsc

_NC, _NS = 2, 16
_C = 32


def kernel(input_embeddings, table):
    B, S, D = input_embeddings.shape
    BS = B * S
    NW = _NC * _NS
    rows_per_w = BS // NW
    chunks = rows_per_w // _C
    CD = _C * D

    mesh = plsc.VectorSubcoreMesh(core_axis_name="c", subcore_axis_name="s")

    @functools.partial(
        pl.kernel,
        mesh=mesh,
        out_type=jax.ShapeDtypeStruct((BS * D,), jnp.float32),
        scratch_types=[
            pltpu.VMEM((CD,), jnp.float32),
            pltpu.VMEM((CD,), jnp.float32),
        ],
    )
    def sc_add(x_hbm, t_hbm, out_hbm, xbuf, tbuf):
        wid = lax.axis_index("s") * _NC + lax.axis_index("c")
        row0 = wid * rows_per_w

        def body(k, carry):
            rbase = row0 + k * _C
            tbase = lax.rem(rbase, S)
            pltpu.sync_copy(x_hbm.at[pl.ds(rbase * D, CD)], xbuf)
            pltpu.sync_copy(t_hbm.at[pl.ds(tbase * D, CD)], tbuf)

            def add_body(i, c):
                sl = pl.ds(i * 16, 16)
                xbuf[sl] = xbuf[sl] + tbuf[sl]
                return c

            lax.fori_loop(0, CD // 16, add_body, 0)
            pltpu.sync_copy(xbuf, out_hbm.at[pl.ds(rbase * D, CD)])
            return carry

        lax.fori_loop(0, chunks, body, 0)

    out = sc_add(input_embeddings.reshape(BS * D), table.reshape(S * D))
    return out.reshape(B, S, D)


# speedup vs baseline: 1.7502x; 1.7502x over previous
import functools
import jax
import jax.numpy as jnp
from jax import lax
from jax.experimental import pallas as pl
from jax.experimental.pallas import tpu as pltpu
from jax.experimental.pallas import tpu_sc as plsc

_NC, _NS = 2, 16
_C = 16


def kernel(input_embeddings, table):
    B, S, D = input_embeddings.shape
    BS = B * S
    NW = _NC * _NS
    rows_per_w = BS // NW
    chunks = rows_per_w // _C
    CD = _C * D

    mesh = plsc.VectorSubcoreMesh(core_axis_name="c", subcore_axis_name="s")

    @functools.partial(
        pl.kernel,
        mesh=mesh,
        out_type=jax.ShapeDtypeStruct((BS * D,), jnp.float32),
        scratch_types=[
            pltpu.VMEM((CD,), jnp.float32),
            pltpu.VMEM((CD,), jnp.float32),
            pltpu.VMEM((CD,), jnp.float32),
            pltpu.VMEM((CD,), jnp.float32),
            pltpu.SemaphoreType.DMA,
            pltpu.SemaphoreType.DMA,
            pltpu.SemaphoreType.DMA,
            pltpu.SemaphoreType.DMA,
        ],
    )
    def sc_add(x_hbm, t_hbm, out_hbm, xb0, xb1, tb0, tb1, is0, is1, os0, os1):
        wid = lax.axis_index("s") * _NC + lax.axis_index("c")
        row0 = wid * rows_per_w
        xbufs, tbufs = (xb0, xb1), (tb0, tb1)
        isems, osems = (is0, is1), (os0, os1)

        def start_in(j, b):
            rbase = (row0 + j * _C) * D
            tbase = lax.rem(row0 + j * _C, S) * D
            pltpu.async_copy(x_hbm.at[pl.ds(rbase, CD)], xbufs[b], isems[b])
            pltpu.async_copy(t_hbm.at[pl.ds(tbase, CD)], tbufs[b], isems[b])

        def wait_in(j, b):
            rbase = (row0 + j * _C) * D
            tbase = lax.rem(row0 + j * _C, S) * D
            pltpu.make_async_copy(
                x_hbm.at[pl.ds(rbase, CD)], xbufs[b], isems[b]).wait()
            pltpu.make_async_copy(
                t_hbm.at[pl.ds(tbase, CD)], tbufs[b], isems[b]).wait()

        def start_out(j, b):
            rbase = (row0 + j * _C) * D
            pltpu.async_copy(xbufs[b], out_hbm.at[pl.ds(rbase, CD)], osems[b])

        def wait_out(j, b):
            rbase = (row0 + j * _C) * D
            pltpu.make_async_copy(
                xbufs[b], out_hbm.at[pl.ds(rbase, CD)], osems[b]).wait()

        start_in(0, 0)

        def half_step(jj, b):
            j = jj * 2 + b
            xb, tb = xbufs[b], tbufs[b]

            @pl.when(j + 1 < chunks)
            def _():
                @pl.when(j >= 1)
                def _():
                    wait_out(j - 1, 1 - b)
                start_in(j + 1, 1 - b)

            wait_in(j, b)

            @plsc.parallel_loop(0, CD, 16, unroll=8)
            def _(i):
                sl = pl.ds(i, 16)
                xb[sl] = xb[sl] + tb[sl]

            start_out(j, b)

        def body(jj, carry):
            half_step(jj, 0)
            half_step(jj, 1)
            return carry

        lax.fori_loop(0, chunks // 2, body, 0)
        wait_out(chunks - 2, 0)
        wait_out(chunks - 1, 1)

    out = sc_add(input_embeddings.reshape(BS * D), table.reshape(S * D))
    return out.reshape(B, S, D)


# speedup vs baseline: 4.4545x; 2.5452x over previous
import functools
import jax
import jax.numpy as jnp
from jax import lax
from jax.experimental import pallas as pl
from jax.experimental.pallas import tpu as pltpu
from jax.experimental.pallas import tpu_sc as plsc

_NC, _NS = 2, 16
_C = 16


def kernel(input_embeddings, table):
    B, S, D = input_embeddings.shape
    BS = B * S
    NW = _NC * _NS
    rows_per_w = BS // NW
    chunks = rows_per_w // _C

    mesh = plsc.VectorSubcoreMesh(core_axis_name="c", subcore_axis_name="s")

    @functools.partial(
        pl.kernel,
        mesh=mesh,
        out_type=jax.ShapeDtypeStruct((BS, D), jnp.float32),
        scratch_types=[
            pltpu.VMEM((_C, D), jnp.float32),
            pltpu.VMEM((_C, D), jnp.float32),
            pltpu.VMEM((_C, D), jnp.float32),
            pltpu.VMEM((_C, D), jnp.float32),
            pltpu.SemaphoreType.DMA,
            pltpu.SemaphoreType.DMA,
            pltpu.SemaphoreType.DMA,
            pltpu.SemaphoreType.DMA,
        ],
        compiler_params=pltpu.CompilerParams(use_tc_tiling_on_sc=True),
    )
    def sc_add(x_hbm, t_hbm, out_hbm, xb0, xb1, tb0, tb1, is0, is1, os0, os1):
        wid = lax.axis_index("s") * _NC + lax.axis_index("c")
        row0 = wid * rows_per_w
        xbufs, tbufs = (xb0, xb1), (tb0, tb1)
        isems, osems = (is0, is1), (os0, os1)

        def start_in(j, b):
            rbase = row0 + j * _C
            tbase = lax.rem(rbase, S)
            pltpu.async_copy(x_hbm.at[pl.ds(rbase, _C)], xbufs[b], isems[b])
            pltpu.async_copy(t_hbm.at[pl.ds(tbase, _C)], tbufs[b], isems[b])

        def wait_in(j, b):
            rbase = row0 + j * _C
            tbase = lax.rem(rbase, S)
            pltpu.make_async_copy(
                x_hbm.at[pl.ds(rbase, _C)], xbufs[b], isems[b]).wait()
            pltpu.make_async_copy(
                t_hbm.at[pl.ds(tbase, _C)], tbufs[b], isems[b]).wait()

        def start_out(j, b):
            rbase = row0 + j * _C
            pltpu.async_copy(xbufs[b], out_hbm.at[pl.ds(rbase, _C)], osems[b])

        def wait_out(j, b):
            rbase = row0 + j * _C
            pltpu.make_async_copy(
                xbufs[b], out_hbm.at[pl.ds(rbase, _C)], osems[b]).wait()

        start_in(0, 0)

        def half_step(jj, b):
            j = jj * 2 + b
            xb, tb = xbufs[b], tbufs[b]

            @pl.when(j + 1 < chunks)
            def _():
                @pl.when(j >= 1)
                def _():
                    wait_out(j - 1, 1 - b)
                start_in(j + 1, 1 - b)

            wait_in(j, b)

            @plsc.parallel_loop(0, D, 16, unroll=2)
            def _(i):
                sl = pl.ds(i, 16)
                for r in range(_C):
                    xb[r, sl] = xb[r, sl] + tb[r, sl]

            start_out(j, b)

        def body(jj, carry):
            half_step(jj, 0)
            half_step(jj, 1)
            return carry

        lax.fori_loop(0, chunks // 2, body, 0)
        wait_out(chunks - 2, 0)
        wait_out(chunks - 1, 1)

    out = sc_add(input_embeddings.reshape(BS, D), table)
    return out.reshape(B, S, D)


# speedup vs baseline: 4.8402x; 1.0866x over previous
import functools
import jax
import jax.numpy as jnp
from jax import lax
from jax.experimental import pallas as pl
from jax.experimental.pallas import tpu as pltpu
from jax.experimental.pallas import tpu_sc as plsc

_NC, _NS = 2, 16
_C = 8


def kernel(input_embeddings, table):
    B, S, D = input_embeddings.shape
    BS = B * S
    NW = _NC * _NS
    s_per_w = S // NW
    chunks = s_per_w // _C
    BC = B * _C

    mesh = plsc.VectorSubcoreMesh(core_axis_name="c", subcore_axis_name="s")

    @functools.partial(
        pl.kernel,
        mesh=mesh,
        out_type=jax.ShapeDtypeStruct((BS, D), jnp.float32),
        scratch_types=[
            pltpu.VMEM((BC, D), jnp.float32),
            pltpu.VMEM((BC, D), jnp.float32),
            pltpu.VMEM((_C, D), jnp.float32),
            pltpu.VMEM((_C, D), jnp.float32),
            pltpu.SemaphoreType.DMA,
            pltpu.SemaphoreType.DMA,
            pltpu.SemaphoreType.DMA,
            pltpu.SemaphoreType.DMA,
        ],
        compiler_params=pltpu.CompilerParams(use_tc_tiling_on_sc=True),
    )
    def sc_add(x_hbm, t_hbm, out_hbm, xb0, xb1, tb0, tb1, is0, is1, os0, os1):
        wid = lax.axis_index("s") * _NC + lax.axis_index("c")
        s0 = wid * s_per_w
        xbufs, tbufs = (xb0, xb1), (tb0, tb1)
        isems, osems = (is0, is1), (os0, os1)

        def start_in(j, b):
            sbase = s0 + j * _C
            pltpu.async_copy(t_hbm.at[pl.ds(sbase, _C)], tbufs[b], isems[b])
            for bb in range(B):
                pltpu.async_copy(
                    x_hbm.at[pl.ds(bb * S + sbase, _C)],
                    xbufs[b].at[pl.ds(bb * _C, _C)], isems[b])

        def wait_in(j, b):
            sbase = s0 + j * _C
            pltpu.make_async_copy(
                t_hbm.at[pl.ds(sbase, _C)], tbufs[b], isems[b]).wait()
            for bb in range(B):
                pltpu.make_async_copy(
                    x_hbm.at[pl.ds(bb * S + sbase, _C)],
                    xbufs[b].at[pl.ds(bb * _C, _C)], isems[b]).wait()

        def start_out(j, b):
            sbase = s0 + j * _C
            for bb in range(B):
                pltpu.async_copy(
                    xbufs[b].at[pl.ds(bb * _C, _C)],
                    out_hbm.at[pl.ds(bb * S + sbase, _C)], osems[b])

        def wait_out(j, b):
            sbase = s0 + j * _C
            for bb in range(B):
                pltpu.make_async_copy(
                    xbufs[b].at[pl.ds(bb * _C, _C)],
                    out_hbm.at[pl.ds(bb * S + sbase, _C)], osems[b]).wait()

        start_in(0, 0)

        def half_step(jj, b):
            j = jj * 2 + b
            xb, tb = xbufs[b], tbufs[b]

            @pl.when(j + 1 < chunks)
            def _():
                @pl.when(j >= 1)
                def _():
                    wait_out(j - 1, 1 - b)
                start_in(j + 1, 1 - b)

            wait_in(j, b)

            @plsc.parallel_loop(0, D, 16, unroll=2)
            def _(i):
                sl = pl.ds(i, 16)
                for r in range(BC):
                    xb[r, sl] = xb[r, sl] + tb[r % _C, sl]

            start_out(j, b)

        def body(jj, carry):
            half_step(jj, 0)
            half_step(jj, 1)
            return carry

        lax.fori_loop(0, chunks // 2, body, 0)
        wait_out(chunks - 2, 0)
        wait_out(chunks - 1, 1)

    out = sc_add(input_embeddings.reshape(BS, D), table)
    return out.reshape(B, S, D)
